# baseline (speedup 1.0000x reference)
import functools

import jax
import jax.numpy as jnp
from jax import lax
from jax.experimental import pallas as pl
from jax.experimental.pallas import tpu as pltpu
from jax.experimental.pallas import tpu_sc as plsc

SCALE = 0.0625
PH = 7
SR = 2
N, C, H, W = 2, 256, 50, 50
KPAD = 1024
NB = PH * PH
NSLOT = 3
NT = NSLOT * NSLOT
NTP = 16
R = KPAD * NB
NC, NS = 2, 16
NW = NC * NS
RPW = R // NW
SB = 16
GH = SB * NT // 2
NSTEP = RPW // SB
CH = 14
NCH = NSTEP // CH
LANES = 16
CCH = C // LANES


def _prep_body(roist_ref, idx_ref, w_ref):
    b = roist_ref[0, :].astype(jnp.int32)
    x1 = roist_ref[1, :] * SCALE - 0.5
    y1 = roist_ref[2, :] * SCALE - 0.5
    x2 = roist_ref[3, :] * SCALE - 0.5
    y2 = roist_ref[4, :] * SCALE - 0.5
    base = b * (H * W)
    bin_h = (y2 - y1) / PH
    bin_w = (x2 - x1) / PH

    def axis_tables(lo, binsz, lim):
        slot_w, slot_i = [], []
        for p in range(PH):
            corners = []
            for s in range(SR):
                coord = lo + binsz * (p + (2 * s + 1) / (2.0 * SR))
                ok = (coord > -1.0) & (coord < lim)
                cc = jnp.clip(coord, 0.0, lim - 1.0)
                c0 = cc.astype(jnp.int32)
                frac = cc - c0.astype(jnp.float32)
                c1 = jnp.minimum(c0 + 1, int(lim) - 1)
                vf = jnp.where(ok, 0.5, 0.0)
                corners.append(((1.0 - frac) * vf, c0))
                corners.append((frac * vf, c1))
            base0 = corners[0][1]
            ws = []
            for a in range(NSLOT):
                acc = jnp.zeros_like(corners[0][0])
                for cw, ci in corners:
                    acc = acc + jnp.where(ci - base0 == a, cw, 0.0)
                ws.append(acc)
            slot_w.append(ws)
            slot_i.append([jnp.minimum(base0 + a, int(lim) - 1)
                           for a in range(NSLOT)])
        return slot_w, slot_i

    ywts, yids = axis_tables(y1, bin_h, float(H))
    xwts, xids = axis_tables(x1, bin_w, float(W))
    zero = jnp.zeros((KPAD,), jnp.float32)
    for py in range(PH):
        for px in range(PH):
            irow0 = (py * PH + px) * NT
            wrow0 = (py * PH + px) * NTP
            for a in range(NSLOT):
                ybase = base + yids[py][a] * W
                for bslot in range(NSLOT):
                    t = a * NSLOT + bslot
                    idx_ref[irow0 + t, :] = ybase + xids[px][bslot]
                    w_ref[wrow0 + t, :] = ywts[py][a] * xwts[px][bslot]
            for t in range(NT, NTP):
                w_ref[wrow0 + t, :] = zero


def _prep(roist):
    return pl.pallas_call(
        _prep_body,
        out_shape=(
            jax.ShapeDtypeStruct((NB * NT, KPAD), jnp.int32),
            jax.ShapeDtypeStruct((NB * NTP, KPAD), jnp.float32),
        ),
    )(roist)


@functools.cache
def _sc_gather_fn():
    mesh = plsc.VectorSubcoreMesh(
        core_axis_name="c", subcore_axis_name="s",
        num_cores=NC, num_subcores=NS)
    return pl.kernel(
        _sc_gather_body,
        out_type=jax.ShapeDtypeStruct((R, C), jnp.float32),
        mesh=mesh,
        scratch_types=[
            pltpu.VMEM((CH * SB * NT,), jnp.int32),
            pltpu.VMEM((CH * SB * NTP,), jnp.float32),
            pltpu.VMEM((2 * GH, C), jnp.float32),
            pltpu.VMEM((2 * GH, C), jnp.float32),
            pltpu.VMEM((SB, C), jnp.float32),
            pltpu.VMEM((SB, C), jnp.float32),
            pltpu.SemaphoreType.DMA,
            pltpu.SemaphoreType.DMA,
            pltpu.SemaphoreType.DMA,
            pltpu.SemaphoreType.DMA,
        ],
    )


def _sc_gather_body(table_hbm, idxf_hbm, w_hbm, out_hbm,
                    idxc, wc, rows0, rows1, acc0, acc1,
                    sg0, sg1, so0, so1):
    wid = lax.axis_index("s") * NC + lax.axis_index("c")
    r_base = wid * RPW

    def chunk(ch, carry):
        c0 = r_base + ch * CH * SB
        pltpu.sync_copy(idxf_hbm.at[pl.ds(c0 * NT, CH * SB * NT)], idxc)
        pltpu.sync_copy(w_hbm.at[pl.ds(c0 * NTP, CH * SB * NTP)], wc)

        def issue(s, buf, sem):
            o = s * SB * NT
            pltpu.make_async_copy(
                table_hbm.at[idxc.at[pl.ds(o, GH)]],
                buf.at[pl.ds(0, GH)], sem).start()
            pltpu.make_async_copy(
                table_hbm.at[idxc.at[pl.ds(o + GH, GH)]],
                buf.at[pl.ds(GH, GH)], sem).start()

        def wait(buf, sem):
            pltpu.make_async_copy(
                table_hbm.at[pl.ds(0, GH)], buf.at[pl.ds(0, GH)],
                sem).wait()
            pltpu.make_async_copy(
                table_hbm.at[pl.ds(0, GH)], buf.at[pl.ds(GH, GH)],
                sem).wait()

        def compute(s, buf, acc_v, osem, ofirst):
            @pl.when(jnp.logical_not(ofirst))
            def _():
                pltpu.make_async_copy(
                    acc_v, out_hbm.at[pl.ds(r_base, SB)], osem).wait()

            def row(i, carry2):
                accs = [jnp.zeros((LANES,), jnp.float32)
                        for _ in range(CCH)]
                w_vec = wc[pl.ds(s * SB * NTP + i * NTP, NTP)]
                for t in range(NT):
                    ws = lax.gather(
                        w_vec, jnp.full((LANES, 1), t, jnp.int32),
                        lax.GatherDimensionNumbers(
                            offset_dims=(), collapsed_slice_dims=(0,),
                            start_index_map=(0,)),
                        (1,), mode=lax.GatherScatterMode.PROMISE_IN_BOUNDS)
                    for cb in range(CCH):
                        accs[cb] = accs[cb] + ws * buf[
                            i * NT + t, pl.ds(cb * LANES, LANES)]
                for cb in range(CCH):
                    acc_v[i, pl.ds(cb * LANES, LANES)] = accs[cb]
                return carry2

            lax.fori_loop(0, SB, row, 0)
            pltpu.make_async_copy(
                acc_v, out_hbm.at[pl.ds(c0 + s * SB, SB)], osem).start()

        issue(0, rows0, sg0)
        first = ch == 0

        def pair(p, carry2):
            s0 = 2 * p
            f = first & (p == 0)
            issue(s0 + 1, rows1, sg1)
            wait(rows0, sg0)
            compute(s0, rows0, acc0, so0, f)
            issue(s0 + 2, rows0, sg0)
            wait(rows1, sg1)
            compute(s0 + 1, rows1, acc1, so1, f)
            return carry2

        lax.fori_loop(0, CH // 2 - 1, pair, 0)
        issue(CH - 1, rows1, sg1)
        wait(rows0, sg0)
        compute(CH - 2, rows0, acc0, so0, False)
        wait(rows1, sg1)
        compute(CH - 1, rows1, acc1, so1, False)
        return carry

    lax.fori_loop(0, NCH, chunk, 0)
    pltpu.make_async_copy(acc0, out_hbm.at[pl.ds(r_base, SB)], so0).wait()
    pltpu.make_async_copy(acc1, out_hbm.at[pl.ds(r_base, SB)], so1).wait()


TRB = 25


def _tr_body(in_ref, out_ref):
    eye = jnp.eye(NB, dtype=jnp.float32)
    out_ref[...] = lax.dot_general(
        in_ref[...], eye, (((1,), (0,)), ((), ())),
        preferred_element_type=jnp.float32)


def _tr(x, k):
    return pl.pallas_call(
        _tr_body,
        grid=(k // TRB,),
        in_specs=[pl.BlockSpec((TRB, NB, C), lambda i: (i, 0, 0))],
        out_specs=pl.BlockSpec((TRB, C, NB), lambda i: (i, 0, 0)),
        out_shape=jax.ShapeDtypeStruct((k, C, NB), jnp.float32),
    )(x)


def kernel(input, rois):
    K = rois.shape[0]
    table = jnp.transpose(input, (0, 2, 3, 1)).reshape(N * H * W, C)
    roisp = jnp.zeros((KPAD, 5), rois.dtype).at[:K].set(rois)
    idx9, w16 = _prep(roisp.T)
    idxf = jnp.transpose(idx9).reshape(R * NT)
    w2 = jnp.transpose(w16).reshape(R * NTP)
    out_rows = _sc_gather_fn()(table, idxf, w2)
    out = _tr(out_rows.reshape(KPAD, NB, C), K)
    return out.reshape(K, C, PH, PH)


# speedup vs baseline: 1.0060x; 1.0060x over previous
import functools

import jax
import jax.numpy as jnp
from jax import lax
from jax.experimental import pallas as pl
from jax.experimental.pallas import tpu as pltpu
from jax.experimental.pallas import tpu_sc as plsc

SCALE = 0.0625
PH = 7
SR = 2
N, C, H, W = 2, 256, 50, 50
KPAD = 1024
NB = PH * PH
NSLOT = 3
NT = NSLOT * NSLOT
NTP = 16
R = KPAD * NB
NC, NS = 2, 16
NW = NC * NS
RPW = R // NW
SB = 16
GH = SB * NT // 2
NSTEP = RPW // SB
CH = 14
NCH = NSTEP // CH
LANES = 16
CCH = C // LANES


def _prep_body(roist_ref, idx_ref, w_ref):
    b = roist_ref[0, :].astype(jnp.int32)
    x1 = roist_ref[1, :] * SCALE - 0.5
    y1 = roist_ref[2, :] * SCALE - 0.5
    x2 = roist_ref[3, :] * SCALE - 0.5
    y2 = roist_ref[4, :] * SCALE - 0.5
    base = b * (H * W)
    bin_h = (y2 - y1) / PH
    bin_w = (x2 - x1) / PH

    def axis_tables(lo, binsz, lim):
        slot_w, slot_i = [], []
        for p in range(PH):
            corners = []
            for s in range(SR):
                coord = lo + binsz * (p + (2 * s + 1) / (2.0 * SR))
                ok = (coord > -1.0) & (coord < lim)
                cc = jnp.clip(coord, 0.0, lim - 1.0)
                c0 = cc.astype(jnp.int32)
                frac = cc - c0.astype(jnp.float32)
                c1 = jnp.minimum(c0 + 1, int(lim) - 1)
                vf = jnp.where(ok, 0.5, 0.0)
                corners.append(((1.0 - frac) * vf, c0))
                corners.append((frac * vf, c1))
            base0 = corners[0][1]
            ws = []
            for a in range(NSLOT):
                acc = jnp.zeros_like(corners[0][0])
                for cw, ci in corners:
                    acc = acc + jnp.where(ci - base0 == a, cw, 0.0)
                ws.append(acc)
            slot_w.append(ws)
            slot_i.append([jnp.minimum(base0 + a, int(lim) - 1)
                           for a in range(NSLOT)])
        return slot_w, slot_i

    ywts, yids = axis_tables(y1, bin_h, float(H))
    xwts, xids = axis_tables(x1, bin_w, float(W))
    zero = jnp.zeros((KPAD,), jnp.float32)
    for py in range(PH):
        for px in range(PH):
            irow0 = (py * PH + px) * NT
            wrow0 = (py * PH + px) * NTP
            for a in range(NSLOT):
                ybase = base + yids[py][a] * W
                for bslot in range(NSLOT):
                    t = a * NSLOT + bslot
                    idx_ref[irow0 + t, :] = ybase + xids[px][bslot]
                    w_ref[wrow0 + t, :] = ywts[py][a] * xwts[px][bslot]
            for t in range(NT, NTP):
                w_ref[wrow0 + t, :] = zero


def _prep(roist):
    return pl.pallas_call(
        _prep_body,
        out_shape=(
            jax.ShapeDtypeStruct((NB * NT, KPAD), jnp.int32),
            jax.ShapeDtypeStruct((NB * NTP, KPAD), jnp.float32),
        ),
    )(roist)


@functools.cache
def _sc_gather_fn():
    mesh = plsc.VectorSubcoreMesh(
        core_axis_name="c", subcore_axis_name="s",
        num_cores=NC, num_subcores=NS)
    return pl.kernel(
        _sc_gather_body,
        out_type=jax.ShapeDtypeStruct((R, C), jnp.float32),
        mesh=mesh,
        scratch_types=[
            pltpu.VMEM((CH * SB * NT,), jnp.int32),
            pltpu.VMEM((CH * SB * NTP,), jnp.float32),
            pltpu.VMEM((2 * GH, C), jnp.float32),
            pltpu.VMEM((2 * GH, C), jnp.float32),
            pltpu.VMEM((SB, C), jnp.float32),
            pltpu.VMEM((SB, C), jnp.float32),
            pltpu.SemaphoreType.DMA,
            pltpu.SemaphoreType.DMA,
            pltpu.SemaphoreType.DMA,
            pltpu.SemaphoreType.DMA,
        ],
    )


def _sc_gather_body(table_hbm, idxf_hbm, w_hbm, out_hbm,
                    idxc, wc, rows0, rows1, acc0, acc1,
                    sg0, sg1, so0, so1):
    wid = lax.axis_index("s") * NC + lax.axis_index("c")
    r_base = wid * RPW

    def chunk(ch, carry):
        c0 = r_base + ch * CH * SB
        pltpu.sync_copy(idxf_hbm.at[pl.ds(c0 * NT, CH * SB * NT)], idxc)
        pltpu.sync_copy(w_hbm.at[pl.ds(c0 * NTP, CH * SB * NTP)], wc)

        def issue(s, buf, sem):
            o = s * SB * NT
            pltpu.make_async_copy(
                table_hbm.at[idxc.at[pl.ds(o, GH)]],
                buf.at[pl.ds(0, GH)], sem).start()
            pltpu.make_async_copy(
                table_hbm.at[idxc.at[pl.ds(o + GH, GH)]],
                buf.at[pl.ds(GH, GH)], sem).start()

        def wait(buf, sem):
            pltpu.make_async_copy(
                table_hbm.at[pl.ds(0, GH)], buf.at[pl.ds(0, GH)],
                sem).wait()
            pltpu.make_async_copy(
                table_hbm.at[pl.ds(0, GH)], buf.at[pl.ds(GH, GH)],
                sem).wait()

        def compute(s, buf, acc_v, osem, ofirst):
            @pl.when(jnp.logical_not(ofirst))
            def _():
                pltpu.make_async_copy(
                    acc_v, out_hbm.at[pl.ds(r_base, SB)], osem).wait()

            def row(i, carry2):
                accs = [jnp.zeros((LANES,), jnp.float32)
                        for _ in range(CCH)]
                w_vec = wc[pl.ds(s * SB * NTP + i * NTP, NTP)]
                for t in range(NT):
                    ws = lax.gather(
                        w_vec, jnp.full((LANES, 1), t, jnp.int32),
                        lax.GatherDimensionNumbers(
                            offset_dims=(), collapsed_slice_dims=(0,),
                            start_index_map=(0,)),
                        (1,), mode=lax.GatherScatterMode.PROMISE_IN_BOUNDS)
                    for cb in range(CCH):
                        accs[cb] = accs[cb] + ws * buf[
                            i * NT + t, pl.ds(cb * LANES, LANES)]
                for cb in range(CCH):
                    acc_v[i, pl.ds(cb * LANES, LANES)] = accs[cb]
                return carry2

            lax.fori_loop(0, SB, row, 0)
            pltpu.make_async_copy(
                acc_v, out_hbm.at[pl.ds(c0 + s * SB, SB)], osem).start()

        issue(0, rows0, sg0)
        first = ch == 0

        def pair(p, carry2):
            s0 = 2 * p
            f = first & (p == 0)
            issue(s0 + 1, rows1, sg1)
            wait(rows0, sg0)
            compute(s0, rows0, acc0, so0, f)
            issue(s0 + 2, rows0, sg0)
            wait(rows1, sg1)
            compute(s0 + 1, rows1, acc1, so1, f)
            return carry2

        lax.fori_loop(0, CH // 2 - 1, pair, 0)
        issue(CH - 1, rows1, sg1)
        wait(rows0, sg0)
        compute(CH - 2, rows0, acc0, so0, False)
        wait(rows1, sg1)
        compute(CH - 1, rows1, acc1, so1, False)
        return carry

    lax.fori_loop(0, NCH, chunk, 0)
    pltpu.make_async_copy(acc0, out_hbm.at[pl.ds(r_base, SB)], so0).wait()
    pltpu.make_async_copy(acc1, out_hbm.at[pl.ds(r_base, SB)], so1).wait()


TRB = 25


def _tr_body(in_ref, out_ref):
    out_ref[...] = jnp.transpose(in_ref[...], (0, 2, 1))


def _tr(x, k):
    return pl.pallas_call(
        _tr_body,
        grid=(k // TRB,),
        in_specs=[pl.BlockSpec((TRB, NB, C), lambda i: (i, 0, 0))],
        out_specs=pl.BlockSpec((TRB, C, NB), lambda i: (i, 0, 0)),
        out_shape=jax.ShapeDtypeStruct((k, C, NB), jnp.float32),
    )(x)


def kernel(input, rois):
    K = rois.shape[0]
    table = jnp.transpose(input, (0, 2, 3, 1)).reshape(N * H * W, C)
    roisp = jnp.zeros((KPAD, 5), rois.dtype).at[:K].set(rois)
    idx9, w16 = _prep(roisp.T)
    idxf = jnp.transpose(idx9).reshape(R * NT)
    w2 = jnp.transpose(w16).reshape(R * NTP)
    out_rows = _sc_gather_fn()(table, idxf, w2)
    out = _tr(out_rows.reshape(KPAD, NB, C), K)
    return out.reshape(K, C, PH, PH)
